# Initial kernel scaffold; baseline (speedup 1.0000x reference)
import jax
import jax.numpy as jnp
from jax.experimental import pallas as pl


def kernel(xyz, x, bn_gamma_0, bn_beta_0, bn_gamma_1, bn_beta_1, bn_gamma_2, bn_beta_2, bn_gamma_3, bn_beta_3):
    raise NotImplementedError("write your pallas kernel here")


# speedup vs baseline: 1.9327x; 1.9327x over previous
import functools
import math

import jax
import jax.numpy as jnp
import numpy as np
from jax.experimental import pallas as pl

B = 8
INPUT_POINTS = 2048
NUM_STAGES = 4
EMBED_DIM = 96
K = 24
SIGMA = 0.3


def _feat_vals(feat_dim, shape, axis):
    j = jax.lax.broadcasted_iota(jnp.int32, shape, axis).astype(jnp.float32)
    return -1.0 + j * (2.0 / feat_dim)


def _embed0_kernel(x_ref, o_ref, *, feat_dim, sigma):
    fv = _feat_vals(feat_dim, (feat_dim, 1), 0)
    parts = []
    for i in range(3):
        xi = x_ref[0, i:i + 1, :]
        e = jnp.exp(-0.5 * (xi - fv) ** 2 / sigma ** 2)
        parts.append(e)
    o_ref[0] = jnp.concatenate(parts, axis=0)


def _embed0(x):
    b, _, n = x.shape
    feat_dim = math.ceil(EMBED_DIM / 3)
    return pl.pallas_call(
        functools.partial(_embed0_kernel, feat_dim=feat_dim, sigma=SIGMA),
        grid=(b,),
        in_specs=[pl.BlockSpec((1, 3, n), lambda i: (i, 0, 0))],
        out_specs=pl.BlockSpec((1, 3 * feat_dim, n), lambda i: (i, 0, 0)),
        out_shape=jax.ShapeDtypeStruct((b, 3 * feat_dim, n), jnp.float32),
    )(x)


def _fps_kernel(xyz_ref, o_ref, *, npoint, n, b):
    x0 = xyz_ref[0]
    x1 = xyz_ref[1]
    x2 = xyz_ref[2]
    lane = jax.lax.broadcasted_iota(jnp.int32, (b, n), 1)
    col = jax.lax.broadcasted_iota(jnp.int32, (b, npoint), 1)

    def body(i, state):
        dist, far, cent = state
        cent = jnp.where(col == i, far[:, None], cent)
        sel = lane == far[:, None]
        c0 = jnp.sum(jnp.where(sel, x0, 0.0), axis=1, keepdims=True)
        c1 = jnp.sum(jnp.where(sel, x1, 0.0), axis=1, keepdims=True)
        c2 = jnp.sum(jnp.where(sel, x2, 0.0), axis=1, keepdims=True)
        d = (x0 - c0) ** 2 + (x1 - c1) ** 2 + (x2 - c2) ** 2
        dist = jnp.minimum(dist, d)
        m = jnp.max(dist, axis=1, keepdims=True)
        far = jnp.min(jnp.where(dist >= m, lane, n), axis=1).astype(jnp.int32)
        return dist, far, cent

    dist0 = jnp.full((b, n), 1e10, dtype=jnp.float32)
    far0 = jnp.zeros((b,), dtype=jnp.int32)
    cent0 = jnp.zeros((b, npoint), dtype=jnp.int32)
    _, _, cent = jax.lax.fori_loop(0, npoint, body, (dist0, far0, cent0))
    o_ref[:, :] = cent


def _fps(xyz, npoint):
    b, n, _ = xyz.shape
    xyz3 = jnp.transpose(xyz, (2, 0, 1))
    return pl.pallas_call(
        functools.partial(_fps_kernel, npoint=npoint, n=n, b=b),
        in_specs=[pl.BlockSpec((3, b, n), lambda: (0, 0, 0))],
        out_specs=pl.BlockSpec((b, npoint), lambda: (0, 0)),
        out_shape=jax.ShapeDtypeStruct((b, npoint), jnp.int32),
    )(xyz3)


def _knn_kernel(q_ref, p_ref, o_ref, *, n, gt, k):
    q = q_ref[0]
    qx = q[:, 0:1]
    qy = q[:, 1:2]
    qz = q[:, 2:3]
    p3 = p_ref[0]
    px = p3[0:1, :]
    py = p3[1:2, :]
    pz = p3[2:3, :]
    sq = qx * qx + qy * qy + qz * qz
    sp = px * px + py * py + pz * pz
    cross = jax.lax.dot_general(
        q.astype(jnp.bfloat16), p3.astype(jnp.bfloat16),
        (((1,), (0,)), ((), ())),
        preferred_element_type=jnp.float32)
    d = sq + sp - 2.0 * cross
    lane = jax.lax.broadcasted_iota(jnp.int32, (gt, n), 1)
    colk = jax.lax.broadcasted_iota(jnp.int32, (gt, k), 1)

    def body(j, state):
        d, idxs = state
        m = jnp.min(d, axis=1, keepdims=True)
        idx = jnp.min(jnp.where(d <= m, lane, n), axis=1).astype(jnp.int32)
        idxs = jnp.where(colk == j, idx[:, None], idxs)
        d = jnp.where(lane == idx[:, None], jnp.float32(3.0e38), d)
        return d, idxs

    idxs0 = jnp.zeros((gt, k), dtype=jnp.int32)
    _, idxs = jax.lax.fori_loop(0, k, body, (d, idxs0))
    o_ref[0] = idxs


def _knn(lc_xyz, xyz):
    b, g, _ = lc_xyz.shape
    n = xyz.shape[1]
    gt = min(g, 128)
    xyz_t = jnp.transpose(xyz, (0, 2, 1))
    return pl.pallas_call(
        functools.partial(_knn_kernel, n=n, gt=gt, k=K),
        grid=(b, g // gt),
        in_specs=[
            pl.BlockSpec((1, gt, 3), lambda i, j: (i, j, 0)),
            pl.BlockSpec((1, 3, n), lambda i, j: (i, 0, 0)),
        ],
        out_specs=pl.BlockSpec((1, gt, K), lambda i, j: (i, j, 0)),
        out_shape=jax.ShapeDtypeStruct((b, g, K), jnp.int32),
    )(lc_xyz, xyz_t)


def _stats_kernel(kxyz_ref, cxyz_ref, kx_ref, cx_ref,
                  s_xyz_ref, ss_xyz_ref, s_x_ref, ss_x_ref):
    dxyz = kxyz_ref[0] - cxyz_ref[0][None]
    dx = kx_ref[0] - cx_ref[0][None]
    s_xyz_ref[0] = jnp.sum(dxyz, axis=(0, 1))[None]
    ss_xyz_ref[0] = jnp.sum(dxyz * dxyz, axis=(0, 1))[None]
    s_x_ref[0] = jnp.sum(dx, axis=(0, 1))[None]
    ss_x_ref[0] = jnp.sum(dx * dx, axis=(0, 1))[None]


def _stats(knn_xyz_t, lc_xyz, knn_x_t, lc_x):
    b, k, g, _ = knn_xyz_t.shape
    c = knn_x_t.shape[-1]
    gt = min(g, 128)
    ngt = g // gt
    vec3 = pl.BlockSpec((1, 1, 3), lambda i, j: (i * ngt + j, 0, 0))
    vecc = pl.BlockSpec((1, 1, c), lambda i, j: (i * ngt + j, 0, 0))
    return pl.pallas_call(
        _stats_kernel,
        grid=(b, ngt),
        in_specs=[
            pl.BlockSpec((1, k, gt, 3), lambda i, j: (i, 0, j, 0)),
            pl.BlockSpec((1, gt, 3), lambda i, j: (i, j, 0)),
            pl.BlockSpec((1, k, gt, c), lambda i, j: (i, 0, j, 0)),
            pl.BlockSpec((1, gt, c), lambda i, j: (i, j, 0)),
        ],
        out_specs=[vec3, vec3, vecc, vecc],
        out_shape=[
            jax.ShapeDtypeStruct((b * ngt, 1, 3), jnp.float32),
            jax.ShapeDtypeStruct((b * ngt, 1, 3), jnp.float32),
            jax.ShapeDtypeStruct((b * ngt, 1, c), jnp.float32),
            jax.ShapeDtypeStruct((b * ngt, 1, c), jnp.float32),
        ],
    )(knn_xyz_t, lc_xyz, knn_x_t, lc_x)


def _agg_kernel(kxyz_ref, cxyz_ref, kx_ref, cx_ref, std_ref, o_ref,
                ps_ref, pss_ref, *, gt, k, c, out_dim, sigma):
    feat_dim = out_dim // 3
    fv = _feat_vals(feat_dim, (1, 1, feat_dim), 2)
    std_xyz = std_ref[0, 0]
    std_x = std_ref[0, 1]

    dxyz = (kxyz_ref[0] - cxyz_ref[0][None]) / (std_xyz + 1e-5)
    parts = []
    for i in range(3):
        di = dxyz[:, :, i:i + 1]
        parts.append(jnp.exp(-0.5 * (di - fv) ** 2 / sigma ** 2))
    pe = jnp.concatenate(parts, axis=-1)

    cx = cx_ref[0]
    xn = (kx_ref[0] - cx[None]) / (std_x + 1e-5)
    x_cat = jnp.concatenate(
        [xn, jnp.broadcast_to(cx[None], (k, gt, c))], axis=-1)
    w = (x_cat + pe) * pe
    agg = jnp.max(w, axis=0) + jnp.sum(w, axis=0) / k
    o_ref[0] = agg
    ps_ref[0, 0] = jnp.sum(agg, axis=0, keepdims=True)
    pss_ref[0, 0] = jnp.sum(agg * agg, axis=0, keepdims=True)


def _agg(knn_xyz_t, lc_xyz, knn_x_t, lc_x, stds, out_dim, gt):
    b, k, g, _ = knn_xyz_t.shape
    c = knn_x_t.shape[-1]
    ngt = g // gt
    kern = functools.partial(_agg_kernel, gt=gt, k=k, c=c,
                             out_dim=out_dim, sigma=SIGMA)
    return pl.pallas_call(
        kern,
        grid=(b, ngt),
        in_specs=[
            pl.BlockSpec((1, k, gt, 3), lambda i, j: (i, 0, j, 0)),
            pl.BlockSpec((1, gt, 3), lambda i, j: (i, j, 0)),
            pl.BlockSpec((1, k, gt, c), lambda i, j: (i, 0, j, 0)),
            pl.BlockSpec((1, gt, c), lambda i, j: (i, j, 0)),
            pl.BlockSpec((1, 2), lambda i, j: (0, 0)),
        ],
        out_specs=[
            pl.BlockSpec((1, gt, out_dim), lambda i, j: (i, j, 0)),
            pl.BlockSpec((1, 1, 1, out_dim), lambda i, j: (i, j, 0, 0)),
            pl.BlockSpec((1, 1, 1, out_dim), lambda i, j: (i, j, 0, 0)),
        ],
        out_shape=[
            jax.ShapeDtypeStruct((b, g, out_dim), jnp.float32),
            jax.ShapeDtypeStruct((b, ngt, 1, out_dim), jnp.float32),
            jax.ShapeDtypeStruct((b, ngt, 1, out_dim), jnp.float32),
        ],
    )(knn_xyz_t, lc_xyz, knn_x_t, lc_x, stds)


def _bn_kernel(a_ref, m_ref, s_ref, g_ref, b_ref, o_ref):
    y = (a_ref[0] - m_ref[0][None]) / s_ref[0][None]
    y = y * g_ref[0][None] + b_ref[0][None]
    o_ref[0] = 0.5 * y * (1.0 + jax.lax.erf(y * np.float32(1.0 / np.sqrt(2.0))))


def _bn_gelu(agg, mean, sqrtv, gamma, beta):
    b, g, c = agg.shape
    vec = pl.BlockSpec((1, c), lambda i: (0, 0))
    return pl.pallas_call(
        _bn_kernel,
        grid=(b,),
        in_specs=[pl.BlockSpec((1, g, c), lambda i: (i, 0, 0)),
                  vec, vec, vec, vec],
        out_specs=pl.BlockSpec((1, g, c), lambda i: (i, 0, 0)),
        out_shape=jax.ShapeDtypeStruct((b, g, c), jnp.float32),
    )(agg, mean[None], sqrtv[None], gamma[None], beta[None])


def _gather_rows(points, idx):
    b = idx.shape[0]
    flat = idx.reshape(b, -1)
    out = jnp.take_along_axis(points, flat[..., None], axis=1)
    return out.reshape(*idx.shape, points.shape[-1])


_AGG_GT = (128, 128, 64, 32)


def kernel(xyz, x, bn_gamma_0, bn_beta_0, bn_gamma_1, bn_beta_1,
           bn_gamma_2, bn_beta_2, bn_gamma_3, bn_beta_3):
    gammas = [bn_gamma_0, bn_gamma_1, bn_gamma_2, bn_gamma_3]
    betas = [bn_beta_0, bn_beta_1, bn_beta_2, bn_beta_3]

    x0 = _embed0(x)
    xyz_list = [xyz]
    x_list = [x0]

    x_t = jnp.transpose(x0, (0, 2, 1))
    out_dim = EMBED_DIM
    for s in range(NUM_STAGES):
        b, n, c = x_t.shape
        out_dim = out_dim * 2
        g = n // 2

        fps_idx = _fps(xyz, g)
        lc_xyz = _gather_rows(xyz, fps_idx)
        lc_x = _gather_rows(x_t, fps_idx)
        knn_idx = _knn(lc_xyz, xyz)
        knn_xyz_t = jnp.transpose(_gather_rows(xyz, knn_idx), (0, 2, 1, 3))
        knn_x_t = jnp.transpose(_gather_rows(x_t, knn_idx), (0, 2, 1, 3))

        s_xyz, ss_xyz, s_x, ss_x = _stats(knn_xyz_t, lc_xyz, knn_x_t, lc_x)
        t0 = jnp.sum(s_xyz)
        t1 = jnp.sum(ss_xyz)
        t2 = jnp.sum(s_x)
        t3 = jnp.sum(ss_x)
        n_xyz = b * g * K * 3
        n_x = b * g * K * c
        var_xyz = (t1 - t0 * t0 / n_xyz) / (n_xyz - 1)
        var_x = (t3 - t2 * t2 / n_x) / (n_x - 1)
        stds = jnp.stack([jnp.sqrt(var_xyz), jnp.sqrt(var_x)])[None]

        agg, ps, pss = _agg(knn_xyz_t, lc_xyz, knn_x_t, lc_x, stds,
                            out_dim, min(g, _AGG_GT[s]))
        cnt = b * g
        mean = jnp.sum(ps, axis=(0, 1, 2)) / cnt
        var = jnp.sum(pss, axis=(0, 1, 2)) / cnt - mean * mean
        sqrtv = jnp.sqrt(var + 1e-5)
        x_out = _bn_gelu(agg, mean, sqrtv, gammas[s], betas[s])

        xyz = lc_xyz
        x_t = x_out
        xyz_list.append(xyz)
        x_list.append(jnp.transpose(x_out, (0, 2, 1)))

    return tuple(xyz_list) + tuple(x_list)
